# Initial kernel scaffold; baseline (speedup 1.0000x reference)
import jax
import jax.numpy as jnp
from jax.experimental import pallas as pl


def kernel(x_in, cl1_w, cl1_b, cl2_w, cl2_b, cl3_w, cl3_b, reg1_w, reg1_b, reg2_w, reg2_b, reg3_w, reg3_b):
    raise NotImplementedError("write your pallas kernel here")


# speedup vs baseline: 23.4752x; 23.4752x over previous
import functools

import jax
import jax.numpy as jnp
from jax.experimental import pallas as pl
from jax.experimental.pallas import tpu as pltpu

CLASSES = 128
SUPER = 8
CF = CLASSES // SUPER
BW = 2048


def _lrelu(v):
    return jnp.where(v > 0, v, 0.01 * v)


def _body(x_ref, cl1_w_ref, cl1_b_ref, cl2_w_ref, cl2_b_ref, cl3_w_ref,
          cl3_b_ref, reg1_w_ref, reg1_b_ref, w2all_ref, b2_ref, w3_ref,
          b3_ref, xreal_ref, mask_ref):
    x = x_ref[0, :, 0, :]
    f32 = jnp.float32

    def mm(w, v):
        return jax.lax.dot_general(w, v, (((1,), (0,)), ((), ())),
                                   preferred_element_type=f32)

    h1 = _lrelu(mm(cl1_w_ref[...], x) + cl1_b_ref[...].reshape(128, 1))
    h2 = _lrelu(mm(cl2_w_ref[...], h1) + cl2_b_ref[...].reshape(128, 1))
    lg = mm(cl3_w_ref[...], h2) + cl3_b_ref[...].reshape(CLASSES + 1, 1)
    mask_ref[0, 0, 0, :] = _lrelu(lg[CLASSES, :])

    cls = lg[0:CLASSES, :]
    m = jnp.max(cls, axis=0, keepdims=True)
    row_iota = jax.lax.broadcasted_iota(jnp.int32, (CLASSES, BW), 0)
    inds = jnp.min(jnp.where(cls == m, row_iota, CLASSES), axis=0,
                   keepdims=True)

    r1 = _lrelu(mm(reg1_w_ref[...], x) + reg1_b_ref[...].reshape(128, 1))
    y = (mm(w2all_ref[0:256, 0:128], r1) +
         mm(w2all_ref[0:256, 128:256], h1))

    s = inds // CF
    x32 = jnp.zeros((32, BW), f32)
    b32 = jnp.zeros((32, BW), f32)
    for e in range(SUPER):
        sel = (s == e)
        x32 = x32 + jnp.where(sel, y[e * 32:(e + 1) * 32, :], 0.0)
        b32 = b32 + jnp.where(sel, b2_ref[...][e, :].reshape(32, 1), 0.0)
    x32 = _lrelu(x32 + b32)

    y3 = mm(w3_ref[...], x32)
    oh = (row_iota == inds)
    reg = jnp.sum(jnp.where(oh, y3 + b3_ref[...].reshape(128, 1), 0.0),
                  axis=0, keepdims=True)
    xreal_ref[0, 0, 0, :] = ((inds.astype(f32) + reg) *
                             (1.0 / float(CLASSES)))[0, :]


@jax.jit
def _run(x_in, cl1_w, cl1_b, cl2_w, cl2_b, cl3_w, cl3_b,
         reg1_w, reg1_b, w2all, b2, w3, b3):
    B, C, H, W = x_in.shape
    grid = (B, W // BW)
    wspec = lambda shape: pl.BlockSpec(shape, lambda b, j: (0,) * len(shape))
    out_shapes = (
        jax.ShapeDtypeStruct((B, 1, 1, W), jnp.float32),
        jax.ShapeDtypeStruct((B, 1, 1, W), jnp.float32),
    )
    ospec = pl.BlockSpec((1, 1, 1, BW), lambda b, j: (b, 0, 0, j))
    return pl.pallas_call(
        _body,
        grid=grid,
        in_specs=[
            pl.BlockSpec((1, C, 1, BW), lambda b, j: (b, 0, 0, j)),
            wspec((128, 128)), wspec((128,)),
            wspec((128, 128)), wspec((128,)),
            wspec((CLASSES + 1, 128)), wspec((CLASSES + 1,)),
            wspec((128, 128)), wspec((128,)),
            wspec((256, 256)), wspec((SUPER, 32)),
            wspec((128, 32)), wspec((128,)),
        ],
        out_specs=(ospec, ospec),
        out_shape=out_shapes,
    )(x_in, cl1_w, cl1_b, cl2_w, cl2_b, cl3_w, cl3_b,
      reg1_w, reg1_b, w2all, b2, w3, b3)


def kernel(x_in, cl1_w, cl1_b, cl2_w, cl2_b, cl3_w, cl3_b,
           reg1_w, reg1_b, reg2_w, reg2_b, reg3_w, reg3_b):
    w2all = jnp.transpose(reg2_w, (0, 2, 1)).reshape(SUPER * 32, 256)
    w3 = reg3_w[:, :, 0]
    b3 = reg3_b[:, 0]
    x_real, mask = _run(x_in, cl1_w, cl1_b, cl2_w, cl2_b, cl3_w, cl3_b,
                        reg1_w, reg1_b, w2all, reg2_b, w3, b3)
    return (x_real, mask)
